# baseline (speedup 1.0000x reference)
import functools
import jax
import jax.numpy as jnp
from jax import lax
from jax.experimental import pallas as pl
from jax.experimental.pallas import tpu as pltpu

_HD = 4096
_S = 2048
_NLAYERS = 3
_THRESH = 0.5
_KEEP = 1843
_LAM = 1.0
_INT_MIN = -2147483648


def _dsum_body(x_ref, o_ref):
    @pl.when(pl.program_id(0) == 0)
    def _():
        o_ref[...] = jnp.zeros_like(o_ref)
    o_ref[...] += jnp.sum(x_ref[...], axis=0, keepdims=True)


def _attn_body(a0_ref, a1_ref, a2_ref, o_ref):
    @pl.when(pl.program_id(0) == 0)
    def _():
        o_ref[...] = jnp.zeros_like(o_ref)
    s = (jnp.sum(a0_ref[...], axis=0, keepdims=True)
         + jnp.sum(a1_ref[...], axis=0, keepdims=True)
         + jnp.sum(a2_ref[...], axis=0, keepdims=True))
    o_ref[...] += s


def _emb_body(e_ref, h_ref, cn_ref, sq_ref):
    e = e_ref[...]
    h = h_ref[...]
    cn_ref[...] = lax.dot_general(
        h, e, (((1,), (1,)), ((), ())),
        preferred_element_type=jnp.float32,
        precision=lax.Precision.HIGHEST)
    ones = jnp.ones_like(h)
    sq_ref[...] = lax.dot_general(
        ones, e * e, (((1,), (1,)), ((), ())),
        preferred_element_type=jnp.float32,
        precision=lax.Precision.HIGHEST)


def _key_i32(x):
    b = lax.bitcast_convert_type(x, jnp.int32)
    return jnp.where(b >= 0, b, jnp.int32(_INT_MIN) - b)


def _final_body(cs_ref, cn_ref, sq_ref, hs_ref, lg_ref, nl_ref,
                cons_ref, core_ref, noise_ref, fl_ref):
    n_attn = jnp.float32(_NLAYERS * 8 * _S)
    agg = cs_ref[...] / n_attn
    a_norm = agg / (jnp.sum(agg) + 1e-8)
    a_scaled = jnp.clip(a_norm * jnp.float32(_S), 0.0, 1.0)

    h_mean = hs_ref[...] / jnp.float32(_S)
    h_norm = jnp.sqrt(jnp.sum(h_mean * h_mean))
    vt_norm = jnp.sqrt(sq_ref[...])
    vdot = cn_ref[...] / jnp.float32(_S)
    cos = vdot / (vt_norm + 1e-8) / (h_norm + 1e-8)
    sem = 0.5 * (cos + 1.0)
    consistency = 0.5 * sem + 0.5 * a_scaled

    key = _key_i32(consistency)
    k = jnp.int32(_KEEP)

    def count_ge(c):
        return jnp.sum((key >= c).astype(jnp.int32))

    m0 = jnp.where(count_ge(jnp.int32(0)) >= k, jnp.int32(0),
                   jnp.int32(_INT_MIN))

    def body(i, m):
        bit = lax.shift_left(jnp.int32(1), jnp.int32(30) - i)
        cand = m + bit
        return jnp.where(count_ge(cand) >= k, cand, m)

    m = lax.fori_loop(0, 31, body, m0)

    core = jnp.logical_or(key >= m, consistency >= jnp.float32(_THRESH))
    core_f = core.astype(jnp.float32)
    cons_ref[...] = consistency
    core_ref[...] = core_f
    noise_ref[...] = 1.0 - core_f
    fl_ref[...] = (1.0 + _LAM) * lg_ref[...] - _LAM * nl_ref[...]


def kernel(input_embeds, decoder_hidden_states, attn_0, attn_1, attn_2,
           logits, noise_logits):
    B, S, D = input_embeds.shape
    H = attn_0.shape[1]
    V = logits.shape[1]

    dec = decoder_hidden_states.reshape(S, D)
    hsum = pl.pallas_call(
        _dsum_body,
        grid=(S // 256,),
        in_specs=[pl.BlockSpec((256, D), lambda i: (i, 0))],
        out_specs=pl.BlockSpec((1, D), lambda i: (0, 0)),
        out_shape=jax.ShapeDtypeStruct((1, D), jnp.float32),
    )(dec)

    rows = H * S
    bq = 512
    a0 = attn_0.reshape(rows, S)
    a1 = attn_1.reshape(rows, S)
    a2 = attn_2.reshape(rows, S)
    colsum = pl.pallas_call(
        _attn_body,
        grid=(rows // bq,),
        in_specs=[pl.BlockSpec((bq, S), lambda i: (i, 0))] * 3,
        out_specs=pl.BlockSpec((1, S), lambda i: (0, 0)),
        out_shape=jax.ShapeDtypeStruct((1, S), jnp.float32),
    )(a0, a1, a2)

    emb = input_embeds.reshape(S, D)
    bs = 256
    cosnum, sqnorm = pl.pallas_call(
        _emb_body,
        grid=(S // bs,),
        in_specs=[
            pl.BlockSpec((bs, D), lambda i: (i, 0)),
            pl.BlockSpec((1, D), lambda i: (0, 0)),
        ],
        out_specs=[
            pl.BlockSpec((1, bs), lambda i: (0, i)),
            pl.BlockSpec((1, bs), lambda i: (0, i)),
        ],
        out_shape=[
            jax.ShapeDtypeStruct((1, S), jnp.float32),
            jax.ShapeDtypeStruct((1, S), jnp.float32),
        ],
    )(emb, hsum)

    cons, core_f, noise_f, final_logits = pl.pallas_call(
        _final_body,
        out_shape=[
            jax.ShapeDtypeStruct((1, S), jnp.float32),
            jax.ShapeDtypeStruct((1, S), jnp.float32),
            jax.ShapeDtypeStruct((1, S), jnp.float32),
            jax.ShapeDtypeStruct((B, V), jnp.float32),
        ],
    )(colsum, cosnum, sqnorm, hsum, logits, noise_logits)

    cons = cons.reshape(B, S)
    core_mask = core_f.reshape(B, S).astype(bool)
    noise_mask = noise_f.reshape(B, S).astype(bool)
    return (cons, cons, core_mask, noise_mask, core_mask, final_logits)


# speedup vs baseline: 1.0188x; 1.0188x over previous
import functools
import jax
import jax.numpy as jnp
from jax import lax
from jax.experimental import pallas as pl
from jax.experimental.pallas import tpu as pltpu

_HD = 4096
_S = 2048
_NLAYERS = 3
_THRESH = 0.5
_KEEP = 1843
_LAM = 1.0
_INT_MIN = -2147483648


def _sums_body(a0_ref, a1_ref, a2_ref, d_ref, cs_ref, hs_ref):
    @pl.when(pl.program_id(0) == 0)
    def _():
        cs_ref[...] = jnp.zeros_like(cs_ref)
        hs_ref[...] = jnp.zeros_like(hs_ref)
    cs_ref[...] += (jnp.sum(a0_ref[...], axis=0, keepdims=True)
                    + jnp.sum(a1_ref[...], axis=0, keepdims=True)
                    + jnp.sum(a2_ref[...], axis=0, keepdims=True))
    hs_ref[...] += jnp.sum(d_ref[...], axis=0, keepdims=True)


def _key_i32(x):
    b = lax.bitcast_convert_type(x, jnp.int32)
    return jnp.where(b >= 0, b, jnp.int32(_INT_MIN) - b)


def _emb_final_body(e_ref, hs_ref, cs_ref, lg_ref, nl_ref,
                    cons_ref, core_ref, noise_ref, fl_ref,
                    cn_acc, sq_acc, *, bs, nsteps):
    i = pl.program_id(0)
    e = e_ref[...]
    h = hs_ref[...]
    cn_acc[0, pl.ds(i * bs, bs)] = lax.dot_general(
        h, e, (((1,), (1,)), ((), ())),
        preferred_element_type=jnp.float32,
        precision=lax.Precision.HIGHEST)[0]
    ones = jnp.ones_like(h)
    sq_acc[0, pl.ds(i * bs, bs)] = lax.dot_general(
        ones, e * e, (((1,), (1,)), ((), ())),
        preferred_element_type=jnp.float32,
        precision=lax.Precision.HIGHEST)[0]

    @pl.when(i == nsteps - 1)
    def _finalize():
        n_attn = jnp.float32(_NLAYERS * 8 * _S)
        agg = cs_ref[...] / n_attn
        a_norm = agg / (jnp.sum(agg) + 1e-8)
        a_scaled = jnp.clip(a_norm * jnp.float32(_S), 0.0, 1.0)

        h_mean = h / jnp.float32(_S)
        h_norm = jnp.sqrt(jnp.sum(h_mean * h_mean))
        vt_norm = jnp.sqrt(sq_acc[...])
        vdot = cn_acc[...] / jnp.float32(_S)
        cos = vdot / (vt_norm + 1e-8) / (h_norm + 1e-8)
        sem = 0.5 * (cos + 1.0)
        consistency = 0.5 * sem + 0.5 * a_scaled

        key = _key_i32(consistency)
        k = jnp.int32(_KEEP)

        def count_ge(c):
            return jnp.sum((key >= c).astype(jnp.int32))

        m0 = jnp.where(count_ge(jnp.int32(0)) >= k, jnp.int32(0),
                       jnp.int32(_INT_MIN))

        def body(j, m):
            bit = lax.shift_left(jnp.int32(1), jnp.int32(30) - j)
            cand = m + bit
            return jnp.where(count_ge(cand) >= k, cand, m)

        m = lax.fori_loop(0, 31, body, m0)

        core = jnp.logical_or(key >= m, consistency >= jnp.float32(_THRESH))
        core_f = core.astype(jnp.float32)
        cons_ref[...] = consistency
        core_ref[...] = core_f
        noise_ref[...] = 1.0 - core_f
        fl_ref[...] = (1.0 + _LAM) * lg_ref[...] - _LAM * nl_ref[...]


def kernel(input_embeds, decoder_hidden_states, attn_0, attn_1, attn_2,
           logits, noise_logits):
    B, S, D = input_embeds.shape
    H = attn_0.shape[1]
    V = logits.shape[1]

    rows = H * S
    bq = 512
    nq = rows // bq
    bd = S // nq
    a0 = attn_0.reshape(rows, S)
    a1 = attn_1.reshape(rows, S)
    a2 = attn_2.reshape(rows, S)
    dec = decoder_hidden_states.reshape(S, D)

    colsum, hsum = pl.pallas_call(
        _sums_body,
        grid=(nq,),
        in_specs=[
            pl.BlockSpec((bq, S), lambda i: (i, 0)),
            pl.BlockSpec((bq, S), lambda i: (i, 0)),
            pl.BlockSpec((bq, S), lambda i: (i, 0)),
            pl.BlockSpec((bd, D), lambda i: (i, 0)),
        ],
        out_specs=[
            pl.BlockSpec((1, S), lambda i: (0, 0)),
            pl.BlockSpec((1, D), lambda i: (0, 0)),
        ],
        out_shape=[
            jax.ShapeDtypeStruct((1, S), jnp.float32),
            jax.ShapeDtypeStruct((1, D), jnp.float32),
        ],
    )(a0, a1, a2, dec)

    emb = input_embeds.reshape(S, D)
    bs = 256
    nsteps = S // bs
    cons, core_f, noise_f, final_logits = pl.pallas_call(
        functools.partial(_emb_final_body, bs=bs, nsteps=nsteps),
        grid=(nsteps,),
        in_specs=[
            pl.BlockSpec((bs, D), lambda i: (i, 0)),
            pl.BlockSpec((1, D), lambda i: (0, 0)),
            pl.BlockSpec((1, S), lambda i: (0, 0)),
            pl.BlockSpec((B, V), lambda i: (0, 0)),
            pl.BlockSpec((B, V), lambda i: (0, 0)),
        ],
        out_specs=[
            pl.BlockSpec((1, S), lambda i: (0, 0)),
            pl.BlockSpec((1, S), lambda i: (0, 0)),
            pl.BlockSpec((1, S), lambda i: (0, 0)),
            pl.BlockSpec((B, V), lambda i: (0, 0)),
        ],
        out_shape=[
            jax.ShapeDtypeStruct((1, S), jnp.float32),
            jax.ShapeDtypeStruct((1, S), jnp.float32),
            jax.ShapeDtypeStruct((1, S), jnp.float32),
            jax.ShapeDtypeStruct((B, V), jnp.float32),
        ],
        scratch_shapes=[
            pltpu.VMEM((1, S), jnp.float32),
            pltpu.VMEM((1, S), jnp.float32),
        ],
    )(emb, hsum, colsum, logits, noise_logits)

    cons = cons.reshape(B, S)
    core_mask = core_f.reshape(B, S).astype(bool)
    noise_mask = noise_f.reshape(B, S).astype(bool)
    return (cons, cons, core_mask, noise_mask, core_mask, final_logits)


# speedup vs baseline: 1.0956x; 1.0753x over previous
import functools
import jax
import jax.numpy as jnp
from jax import lax
from jax.experimental import pallas as pl
from jax.experimental.pallas import tpu as pltpu
from jax.experimental.pallas import tpu_sc as plsc

_HD = 4096
_S = 2048
_NLAYERS = 3
_THRESH = 0.5
_KEEP = 1843
_LAM = 1.0
_INT_MIN = -2147483648

_NC = 2
_NS = 16
_LN = 16


def _attn_body(a0_ref, a1_ref, a2_ref, cs_ref):
    @pl.when(pl.program_id(0) == 0)
    def _():
        cs_ref[...] = jnp.zeros_like(cs_ref)
    cs_ref[...] += (jnp.sum(a0_ref[...], axis=0, keepdims=True)
                    + jnp.sum(a1_ref[...], axis=0, keepdims=True)
                    + jnp.sum(a2_ref[...], axis=0, keepdims=True))


_DCH = 64
_ECH = 8
_EROWS = _S // (_NC * _NS)


def _sc_body(dec, emb, hsum_o, cn_o, sq_o,
             dbuf, ebuf, accv, hbuf, cnv, sqv, shsum, sem_a, sem_b):
    cid = lax.axis_index("c")
    sid = lax.axis_index("s")
    sems = [sem_a, sem_b]

    col0 = sid * (_HD // _NS)
    ncols = _HD // _NS
    nch = _S // _DCH
    cps = [None, None]
    cps[0] = pltpu.make_async_copy(
        dec.at[pl.ds(0, _DCH), pl.ds(col0, ncols)], dbuf.at[0], sem_a)
    cps[0].start()
    acc = tuple(jnp.zeros((_LN,), jnp.float32) for _ in range(ncols // _LN))
    for c in range(nch):
        cur = c & 1
        if c + 1 < nch:
            nxt = (c + 1) & 1
            cps[nxt] = pltpu.make_async_copy(
                dec.at[pl.ds((c + 1) * _DCH, _DCH), pl.ds(col0, ncols)],
                dbuf.at[nxt], sems[nxt])
            cps[nxt].start()
        cps[cur].wait()

        def rbody(r, a, cur=cur):
            return tuple(a[j] + dbuf[cur, r, pl.ds(j * _LN, _LN)]
                         for j in range(ncols // _LN))
        acc = lax.fori_loop(0, _DCH, rbody, acc)

    for j in range(ncols // _LN):
        accv[pl.ds(j * _LN, _LN)] = acc[j]
    pltpu.sync_copy(accv, shsum.at[pl.ds(col0, ncols)])
    plsc.subcore_barrier()
    pltpu.sync_copy(shsum, hbuf)

    @pl.when(jnp.logical_and(cid == 0, sid == 0))
    def _():
        pltpu.sync_copy(hbuf, hsum_o)

    wid = sid * _NC + cid
    row0 = wid * _EROWS
    nech = _EROWS // _ECH
    eps = [None, None]
    eps[0] = pltpu.make_async_copy(
        emb.at[pl.ds(row0, _ECH)], ebuf.at[0], sem_a)
    eps[0].start()
    lane_iota = lax.iota(jnp.int32, _LN)
    for c in range(nech):
        cur = c & 1
        if c + 1 < nech:
            nxt = (c + 1) & 1
            eps[nxt] = pltpu.make_async_copy(
                emb.at[pl.ds(row0 + (c + 1) * _ECH, _ECH)],
                ebuf.at[nxt], sems[nxt])
            eps[nxt].start()
        eps[cur].wait()
        for r in range(_ECH):
            def jbody(j, cs, cur=cur, r=r):
                cn0, cn1, sq0, sq1 = cs
                base = j * (8 * _LN)
                for u in range(8):
                    off = base + u * _LN
                    h_v = hbuf[pl.ds(off, _LN)]
                    e_v = ebuf[cur, r, pl.ds(off, _LN)]
                    if u & 1:
                        cn1 = cn1 + e_v * h_v
                        sq1 = sq1 + e_v * e_v
                    else:
                        cn0 = cn0 + e_v * h_v
                        sq0 = sq0 + e_v * e_v
                return (cn0, cn1, sq0, sq1)

            z = jnp.zeros((_LN,), jnp.float32)
            cn0, cn1, sq0, sq1 = lax.fori_loop(
                0, _HD // (8 * _LN), jbody, (z, z, z, z))
            rl = jnp.full((_LN,), c * _ECH + r, jnp.int32)
            plsc.store_scatter(cnv, [lane_iota, rl], cn0 + cn1)
            plsc.store_scatter(sqv, [lane_iota, rl], sq0 + sq1)
    outcps = []
    for j in range(_LN):
        outcps.append(pltpu.make_async_copy(
            cnv.at[j], cn_o.at[pl.ds(j * _S + row0, _EROWS)], sem_a))
        outcps.append(pltpu.make_async_copy(
            sqv.at[j], sq_o.at[pl.ds(j * _S + row0, _EROWS)], sem_b))
    for cp in outcps:
        cp.start()
    for cp in outcps:
        cp.wait()


_sc_call = functools.partial(
    pl.kernel,
    mesh=plsc.VectorSubcoreMesh(core_axis_name="c", subcore_axis_name="s"),
    compiler_params=pltpu.CompilerParams(needs_layout_passes=False),
    out_type=[
        jax.ShapeDtypeStruct((_HD,), jnp.float32),
        jax.ShapeDtypeStruct((_LN * _S,), jnp.float32),
        jax.ShapeDtypeStruct((_LN * _S,), jnp.float32),
    ],
    scratch_types=[
        pltpu.VMEM((2, _DCH, _HD // _NS), jnp.float32),
        pltpu.VMEM((2, _ECH, _HD), jnp.float32),
        pltpu.VMEM((_HD // _NS,), jnp.float32),
        pltpu.VMEM((_HD,), jnp.float32),
        pltpu.VMEM((_LN, _EROWS), jnp.float32),
        pltpu.VMEM((_LN, _EROWS), jnp.float32),
        pltpu.VMEM_SHARED((_HD,), jnp.float32),
        pltpu.SemaphoreType.DMA,
        pltpu.SemaphoreType.DMA,
    ],
)(_sc_body)


def _key_i32(x):
    b = lax.bitcast_convert_type(x, jnp.int32)
    return jnp.where(b >= 0, b, jnp.int32(_INT_MIN) - b)


def _final_body(cs_ref, cn_ref, sq_ref, hs_ref, lg_ref, nl_ref,
                cons_ref, core_ref, noise_ref, fl_ref):
    n_attn = jnp.float32(_NLAYERS * 8 * _S)
    agg = cs_ref[...] / n_attn
    a_norm = agg / (jnp.sum(agg) + 1e-8)
    a_scaled = jnp.clip(a_norm * jnp.float32(_S), 0.0, 1.0)

    h_mean = hs_ref[...] / jnp.float32(_S)
    h_norm = jnp.sqrt(jnp.sum(h_mean * h_mean))
    sq = jnp.sum(sq_ref[...], axis=0, keepdims=True)
    vt_norm = jnp.sqrt(sq)
    vdot = jnp.sum(cn_ref[...], axis=0, keepdims=True) / jnp.float32(_S)
    cos = vdot / (vt_norm + 1e-8) / (h_norm + 1e-8)
    sem = 0.5 * (cos + 1.0)
    consistency = 0.5 * sem + 0.5 * a_scaled

    key = _key_i32(consistency)
    k = jnp.int32(_KEEP)

    def count_ge(c):
        return jnp.sum((key >= c).astype(jnp.int32))

    m0 = jnp.where(count_ge(jnp.int32(0)) >= k, jnp.int32(0),
                   jnp.int32(_INT_MIN))

    def body(j, m):
        bit = lax.shift_left(jnp.int32(1), jnp.int32(30) - j)
        cand = m + bit
        return jnp.where(count_ge(cand) >= k, cand, m)

    m = lax.fori_loop(0, 31, body, m0)

    core = jnp.logical_or(key >= m, consistency >= jnp.float32(_THRESH))
    core_f = core.astype(jnp.float32)
    cons_ref[...] = consistency
    core_ref[...] = core_f
    noise_ref[...] = 1.0 - core_f
    fl_ref[...] = (1.0 + _LAM) * lg_ref[...] - _LAM * nl_ref[...]


def kernel(input_embeds, decoder_hidden_states, attn_0, attn_1, attn_2,
           logits, noise_logits):
    B, S, D = input_embeds.shape
    H = attn_0.shape[1]
    V = logits.shape[1]

    rows = H * S
    bq = 512
    nq = rows // bq
    a0 = attn_0.reshape(rows, S)
    a1 = attn_1.reshape(rows, S)
    a2 = attn_2.reshape(rows, S)

    colsum = pl.pallas_call(
        _attn_body,
        grid=(nq,),
        in_specs=[pl.BlockSpec((bq, S), lambda i: (i, 0))] * 3,
        out_specs=pl.BlockSpec((1, S), lambda i: (0, 0)),
        out_shape=jax.ShapeDtypeStruct((1, S), jnp.float32),
    )(a0, a1, a2)

    dec = decoder_hidden_states.reshape(S, D)
    emb = input_embeds.reshape(S, D)
    hsum, cosnum, sqnorm = _sc_call(dec, emb)

    cons, core_f, noise_f, final_logits = pl.pallas_call(
        _final_body,
        out_shape=[
            jax.ShapeDtypeStruct((1, S), jnp.float32),
            jax.ShapeDtypeStruct((1, S), jnp.float32),
            jax.ShapeDtypeStruct((1, S), jnp.float32),
            jax.ShapeDtypeStruct((B, V), jnp.float32),
        ],
    )(colsum, cosnum.reshape(_LN, S), sqnorm.reshape(_LN, S),
      hsum.reshape(1, D), logits, noise_logits)

    cons = cons.reshape(B, S)
    core_mask = core_f.reshape(B, S).astype(bool)
    noise_mask = noise_f.reshape(B, S).astype(bool)
    return (cons, cons, core_mask, noise_mask, core_mask, final_logits)
